# baseline (speedup 1.0000x reference)
import functools

import jax
import jax.numpy as jnp
from jax import lax
from jax.experimental import pallas as pl
from jax.experimental.pallas import tpu as pltpu
from jax.experimental.pallas import tpu_sc as plsc

NUM_ROWS = 100005
EMBED = 32
BATCH = 16384
HIST = 200
NC = 2
NS = 16
NW = NC * NS
CB = 1024
NBC = BATCH // CB
UNITS = HIST * NBC
UPW = UNITS // NW

_mesh = plsc.VectorSubcoreMesh(core_axis_name="c", subcore_axis_name="s")


@functools.partial(
    pl.kernel,
    mesh=_mesh,
    compiler_params=pltpu.CompilerParams(
        use_tc_tiling_on_sc=False, needs_layout_passes=False
    ),
    out_type=jax.ShapeDtypeStruct((HIST, EMBED, BATCH), jnp.float32),
    scratch_types=[
        pltpu.VMEM((2, CB), jnp.int32),
        pltpu.VMEM((2, CB, EMBED), jnp.float32),
        pltpu.VMEM((EMBED, CB), jnp.float32),
        pltpu.SemaphoreType.DMA((2,)),
    ],
)
def _emb_lookup(xt_hbm, table_hbm, out_hbm, idx_v, rows_v, trows_v, gsem):
    wid = lax.axis_index("s") * NC + lax.axis_index("c")
    u0 = wid * UPW
    lane = lax.iota(jnp.int32, 16)

    def unit_hb(u):
        return u >> 4, (u & 15) * CB

    def fire(b, u):
        h, b0 = unit_hb(u)
        pltpu.sync_copy(xt_hbm.at[h, pl.ds(b0, CB)], idx_v.at[b])
        pltpu.async_copy(table_hbm.at[idx_v.at[b]], rows_v.at[b], gsem.at[b])

    def drain_transpose_store(b, u):
        h, b0 = unit_hb(u)
        pltpu.make_async_copy(
            table_hbm.at[pl.ds(0, CB)], rows_v.at[b], gsem.at[b]
        ).wait()

        def tbody(j, carry):
            row_idx = lane + j * 16
            for e in range(EMBED):
                col_idx = jnp.full((16,), e, jnp.int32)
                trows_v[e, pl.ds(j * 16, 16)] = plsc.load_gather(
                    rows_v.at[b], [row_idx, col_idx]
                )
            return carry

        lax.fori_loop(0, CB // 16, tbody, 0)
        pltpu.sync_copy(trows_v, out_hbm.at[h, :, pl.ds(b0, CB)])

    fire(0, u0)
    fire(1, u0 + 1)

    def body(j, carry):
        u = u0 + 2 * j
        for b in range(2):
            drain_transpose_store(b, u + b)
            fire(b, u + b + 2)
        return carry

    lax.fori_loop(0, UPW // 2 - 1, body, 0)
    drain_transpose_store(0, u0 + UPW - 2)
    drain_transpose_store(1, u0 + UPW - 1)


def kernel(x, table):
    xt = jnp.transpose(x.astype(jnp.int32))
    out = _emb_lookup(xt, table)
    return jnp.transpose(out, (2, 0, 1))


# speedup vs baseline: 2.2242x; 2.2242x over previous
import functools

import jax
import jax.numpy as jnp
from jax import lax
from jax.experimental import pallas as pl
from jax.experimental.pallas import tpu as pltpu
from jax.experimental.pallas import tpu_sc as plsc

NUM_ROWS = 100005
EMBED = 32
BATCH = 16384
HIST = 200
NC = 2
NS = 16
NW = NC * NS
CB = 1024
NBC = BATCH // CB
UNITS = HIST * NBC
UPW = UNITS // NW

_mesh = plsc.VectorSubcoreMesh(core_axis_name="c", subcore_axis_name="s")


@functools.partial(
    pl.kernel,
    mesh=_mesh,
    compiler_params=pltpu.CompilerParams(
        use_tc_tiling_on_sc=False, needs_layout_passes=False
    ),
    out_type=jax.ShapeDtypeStruct((HIST, EMBED, BATCH), jnp.float32),
    scratch_types=[
        pltpu.VMEM((2, CB), jnp.int32),
        pltpu.VMEM((2, CB, EMBED), jnp.float32),
        pltpu.VMEM((EMBED, CB + 1), jnp.float32),
        pltpu.SemaphoreType.DMA((2,)),
    ],
)
def _emb_lookup(xt_hbm, table_hbm, out_hbm, idx_v, rows_v, trows_v, gsem):
    wid = lax.axis_index("s") * NC + lax.axis_index("c")
    u0 = wid * UPW
    lane = lax.iota(jnp.int32, 16)

    def unit_hb(u):
        return u >> 4, (u & 15) * CB

    def fire(b, u):
        h, b0 = unit_hb(u)
        pltpu.sync_copy(xt_hbm.at[h, pl.ds(b0, CB)], idx_v.at[b])
        pltpu.async_copy(table_hbm.at[idx_v.at[b]], rows_v.at[b], gsem.at[b])

    def drain_transpose_store(b, u):
        h, b0 = unit_hb(u)
        pltpu.make_async_copy(
            table_hbm.at[pl.ds(0, CB)], rows_v.at[b], gsem.at[b]
        ).wait()

        def tbody(j, carry):
            for r in range(16):
                row = j * 16 + r
                col_idx = jnp.full((16,), row, jnp.int32)
                v0 = rows_v[b, row, pl.ds(0, 16)]
                plsc.store_scatter(trows_v, [lane, col_idx], v0)
                v1 = rows_v[b, row, pl.ds(16, 16)]
                plsc.store_scatter(trows_v, [lane + 16, col_idx], v1)
            return carry

        lax.fori_loop(0, CB // 16, tbody, 0)
        pltpu.sync_copy(
            trows_v.at[:, pl.ds(0, CB)], out_hbm.at[h, :, pl.ds(b0, CB)]
        )

    fire(0, u0)
    fire(1, u0 + 1)

    def body(j, carry):
        u = u0 + 2 * j
        for b in range(2):
            drain_transpose_store(b, u + b)
            fire(b, u + b + 2)
        return carry

    lax.fori_loop(0, UPW // 2 - 1, body, 0)
    drain_transpose_store(0, u0 + UPW - 2)
    drain_transpose_store(1, u0 + UPW - 1)


def kernel(x, table):
    xt = jnp.transpose(x.astype(jnp.int32))
    out = _emb_lookup(xt, table)
    return jnp.transpose(out, (2, 0, 1))


# speedup vs baseline: 2.2928x; 1.0308x over previous
import functools

import jax
import jax.numpy as jnp
from jax import lax
from jax.experimental import pallas as pl
from jax.experimental.pallas import tpu as pltpu
from jax.experimental.pallas import tpu_sc as plsc

NUM_ROWS = 100005
EMBED = 32
BATCH = 16384
HIST = 200
NC = 2
NS = 16
NW = NC * NS
CB = 512
NBC = BATCH // CB
UNITS = HIST * NBC
UPW = UNITS // NW
BSH = NBC.bit_length() - 1

_mesh = plsc.VectorSubcoreMesh(core_axis_name="c", subcore_axis_name="s")


@functools.partial(
    pl.kernel,
    mesh=_mesh,
    compiler_params=pltpu.CompilerParams(
        use_tc_tiling_on_sc=False, needs_layout_passes=False
    ),
    out_type=jax.ShapeDtypeStruct((HIST, EMBED, BATCH), jnp.float32),
    scratch_types=[
        pltpu.VMEM((2, CB), jnp.int32),
        pltpu.VMEM((2, CB, EMBED), jnp.float32),
        pltpu.VMEM((2, EMBED, CB + 1), jnp.float32),
        pltpu.SemaphoreType.DMA((2,)),
        pltpu.SemaphoreType.DMA((2,)),
    ],
)
def _emb_lookup(xt_hbm, table_hbm, out_hbm, idx_v, rows_v, trows_v, gsem, ssem):
    wid = lax.axis_index("s") * NC + lax.axis_index("c")
    u0 = wid * UPW
    lane = lax.iota(jnp.int32, 16)

    def unit_hb(u):
        return u >> BSH, (u & (NBC - 1)) * CB

    def fire(b, u):
        h, b0 = unit_hb(u)
        pltpu.sync_copy(xt_hbm.at[h, pl.ds(b0, CB)], idx_v.at[b])
        pltpu.async_copy(table_hbm.at[idx_v.at[b]], rows_v.at[b], gsem.at[b])

    def drain_gathers(b):
        pltpu.make_async_copy(
            table_hbm.at[pl.ds(0, CB)], rows_v.at[b], gsem.at[b]
        ).wait()

    def transpose(b):
        def tbody(j, carry):
            for r in range(16):
                row = j * 16 + r
                col_idx = jnp.full((16,), row, jnp.int32)
                v0 = rows_v[b, row, pl.ds(0, 16)]
                plsc.store_scatter(trows_v.at[b], [lane, col_idx], v0)
                v1 = rows_v[b, row, pl.ds(16, 16)]
                plsc.store_scatter(trows_v.at[b], [lane + 16, col_idx], v1)
            return carry

        lax.fori_loop(0, CB // 16, tbody, 0)

    def fire_store(b, u):
        h, b0 = unit_hb(u)
        pltpu.async_copy(
            trows_v.at[b, :, pl.ds(0, CB)],
            out_hbm.at[h, :, pl.ds(b0, CB)],
            ssem.at[b],
        )

    def wait_store(b):
        pltpu.make_async_copy(
            trows_v.at[b, :, pl.ds(0, CB)],
            out_hbm.at[0, :, pl.ds(0, CB)],
            ssem.at[b],
        ).wait()

    fire(0, u0)
    fire(1, u0 + 1)
    drain_gathers(0)
    transpose(0)
    fire_store(0, u0)
    fire(0, u0 + 2)
    drain_gathers(1)
    transpose(1)
    fire_store(1, u0 + 1)
    fire(1, u0 + 3)

    def body(j, carry):
        u = u0 + 2 * j
        for b in range(2):
            drain_gathers(b)
            wait_store(b)
            transpose(b)
            fire_store(b, u + b)
            fire(b, u + b + 2)
        return carry

    lax.fori_loop(1, UPW // 2 - 1, body, 0)

    for b in range(2):
        u = u0 + UPW - 2 + b
        drain_gathers(b)
        wait_store(b)
        transpose(b)
        fire_store(b, u)
    wait_store(0)
    wait_store(1)


def kernel(x, table):
    xt = jnp.transpose(x.astype(jnp.int32))
    out = _emb_lookup(xt, table)
    return jnp.transpose(out, (2, 0, 1))


# speedup vs baseline: 2.9893x; 1.3038x over previous
import functools

import jax
import jax.numpy as jnp
from jax import lax
from jax.experimental import pallas as pl
from jax.experimental.pallas import tpu as pltpu
from jax.experimental.pallas import tpu_sc as plsc

NUM_ROWS = 100005
EMBED = 32
BATCH = 16384
HIST = 200
NC = 2
NS = 16
NW = NC * NS
CB = 512
NBC = BATCH // CB
UNITS = HIST * NBC
UPW = UNITS // NW
BSH = NBC.bit_length() - 1

_mesh = plsc.VectorSubcoreMesh(core_axis_name="c", subcore_axis_name="s")


@functools.partial(
    pl.kernel,
    mesh=_mesh,
    compiler_params=pltpu.CompilerParams(
        use_tc_tiling_on_sc=False, needs_layout_passes=False
    ),
    out_type=jax.ShapeDtypeStruct(
        (HIST, EMBED // 8, BATCH // 128, 8, 128), jnp.float32
    ),
    scratch_types=[
        pltpu.VMEM((2, CB), jnp.int32),
        pltpu.VMEM((2, CB, EMBED), jnp.float32),
        pltpu.VMEM((2, EMBED // 8, CB // 128, 8, 129), jnp.float32),
        pltpu.SemaphoreType.DMA((2,)),
        pltpu.SemaphoreType.DMA((2,)),
    ],
)
def _emb_lookup(xt_hbm, table_hbm, out_hbm, idx_v, rows_v, trows_v, gsem, ssem):
    wid = lax.axis_index("s") * NC + lax.axis_index("c")
    u0 = wid * UPW
    lane = lax.iota(jnp.int32, 16)

    def unit_hb(u):
        return u >> BSH, (u & (NBC - 1)) * CB

    def fire(b, u):
        h, b0 = unit_hb(u)
        pltpu.sync_copy(xt_hbm.at[h, pl.ds(b0, CB)], idx_v.at[b])
        pltpu.async_copy(table_hbm.at[idx_v.at[b]], rows_v.at[b], gsem.at[b])

    def drain_gathers(b):
        pltpu.make_async_copy(
            table_hbm.at[pl.ds(0, CB)], rows_v.at[b], gsem.at[b]
        ).wait()

    eb_lo = lane >> 3
    eb_hi = eb_lo + 2
    ei = lane & 7

    def transpose(b):
        def tbody(j, carry):
            for r in range(16):
                row = j * 16 + r
                jb = jnp.full((16,), row >> 7, jnp.int32)
                jc = jnp.full((16,), row & 127, jnp.int32)
                v0 = rows_v[b, row, pl.ds(0, 16)]
                plsc.store_scatter(trows_v.at[b], [eb_lo, jb, ei, jc], v0)
                v1 = rows_v[b, row, pl.ds(16, 16)]
                plsc.store_scatter(trows_v.at[b], [eb_hi, jb, ei, jc], v1)
            return carry

        lax.fori_loop(0, CB // 16, tbody, 0)

    def fire_store(b, u):
        h, b0 = unit_hb(u)
        pltpu.async_copy(
            trows_v.at[b, :, :, :, pl.ds(0, 128)],
            out_hbm.at[h, :, pl.ds(b0 // 128, CB // 128)],
            ssem.at[b],
        )

    def wait_store(b):
        pltpu.make_async_copy(
            trows_v.at[b, :, :, :, pl.ds(0, 128)],
            out_hbm.at[0, :, pl.ds(0, CB // 128)],
            ssem.at[b],
        ).wait()

    fire(0, u0)
    fire(1, u0 + 1)
    drain_gathers(0)
    transpose(0)
    fire_store(0, u0)
    fire(0, u0 + 2)
    drain_gathers(1)
    transpose(1)
    fire_store(1, u0 + 1)
    fire(1, u0 + 3)

    def body(j, carry):
        u = u0 + 2 * j
        for b in range(2):
            drain_gathers(b)
            wait_store(b)
            transpose(b)
            fire_store(b, u + b)
            fire(b, u + b + 2)
        return carry

    lax.fori_loop(1, UPW // 2 - 1, body, 0)

    for b in range(2):
        u = u0 + UPW - 2 + b
        drain_gathers(b)
        wait_store(b)
        transpose(b)
        fire_store(b, u)
    wait_store(0)
    wait_store(1)


def kernel(x, table):
    xt = jnp.transpose(x.astype(jnp.int32))
    out5 = _emb_lookup(xt, table)
    out = jnp.transpose(out5, (2, 4, 0, 1, 3))
    return out.reshape(BATCH, HIST, EMBED)


# speedup vs baseline: 2.9949x; 1.0019x over previous
import functools

import jax
import jax.numpy as jnp
from jax import lax
from jax.experimental import pallas as pl
from jax.experimental.pallas import tpu as pltpu
from jax.experimental.pallas import tpu_sc as plsc

NUM_ROWS = 100005
EMBED = 32
BATCH = 16384
HIST = 200
NC = 2
NS = 16
NW = NC * NS
CB = 512
NBC = BATCH // CB
UNITS = HIST * NBC
UPW = UNITS // NW
BSH = NBC.bit_length() - 1

_mesh = plsc.VectorSubcoreMesh(core_axis_name="c", subcore_axis_name="s")


@functools.partial(
    pl.kernel,
    mesh=_mesh,
    compiler_params=pltpu.CompilerParams(
        use_tc_tiling_on_sc=False, needs_layout_passes=False
    ),
    out_type=jax.ShapeDtypeStruct(
        (HIST, EMBED // 8, BATCH // 128, 8, 128), jnp.float32
    ),
    scratch_types=[
        pltpu.VMEM((2, CB), jnp.int32),
        pltpu.VMEM((2, CB, EMBED), jnp.float32),
        pltpu.VMEM((2, EMBED // 8, CB // 128 + 1, 8, 129), jnp.float32),
        pltpu.SemaphoreType.DMA((2,)),
        pltpu.SemaphoreType.DMA((2,)),
    ],
)
def _emb_lookup(xt_hbm, table_hbm, out_hbm, idx_v, rows_v, trows_v, gsem, ssem):
    wid = lax.axis_index("s") * NC + lax.axis_index("c")
    u0 = wid * UPW
    lane = lax.iota(jnp.int32, 16)

    def unit_hb(u):
        return u >> BSH, (u & (NBC - 1)) * CB

    def fire(b, u):
        h, b0 = unit_hb(u)
        pltpu.sync_copy(xt_hbm.at[h, pl.ds(b0, CB)], idx_v.at[b])
        pltpu.async_copy(table_hbm.at[idx_v.at[b]], rows_v.at[b], gsem.at[b])

    def drain_gathers(b):
        pltpu.make_async_copy(
            table_hbm.at[pl.ds(0, CB)], rows_v.at[b], gsem.at[b]
        ).wait()

    eb_lo = lane >> 3
    eb_hi = eb_lo + 2
    ei = lane & 7

    def transpose(b):
        def tbody(j, carry):
            jb = jnp.full((16,), j >> 3, jnp.int32)
            jc0 = jnp.full((16,), (j * 16) & 127, jnp.int32)
            for r in range(16):
                row = j * 16 + r
                jc = jc0 + r
                v0 = rows_v[b, row, pl.ds(0, 16)]
                plsc.store_scatter(trows_v.at[b], [eb_lo, jb, ei, jc], v0)
                v1 = rows_v[b, row, pl.ds(16, 16)]
                plsc.store_scatter(trows_v.at[b], [eb_hi, jb, ei, jc], v1)
            return carry

        lax.fori_loop(0, CB // 16, tbody, 0)

    def fire_store(b, u):
        h, b0 = unit_hb(u)
        pltpu.async_copy(
            trows_v.at[b, :, pl.ds(0, CB // 128), :, pl.ds(0, 128)],
            out_hbm.at[h, :, pl.ds(b0 // 128, CB // 128)],
            ssem.at[b],
        )

    def wait_store(b):
        pltpu.make_async_copy(
            trows_v.at[b, :, pl.ds(0, CB // 128), :, pl.ds(0, 128)],
            out_hbm.at[0, :, pl.ds(0, CB // 128)],
            ssem.at[b],
        ).wait()

    fire(0, u0)
    fire(1, u0 + 1)
    drain_gathers(0)
    transpose(0)
    fire_store(0, u0)
    fire(0, u0 + 2)
    drain_gathers(1)
    transpose(1)
    fire_store(1, u0 + 1)
    fire(1, u0 + 3)

    def body(j, carry):
        u = u0 + 2 * j
        for b in range(2):
            drain_gathers(b)
            wait_store(b)
            transpose(b)
            fire_store(b, u + b)
            fire(b, u + b + 2)
        return carry

    lax.fori_loop(1, UPW // 2 - 1, body, 0)

    for b in range(2):
        u = u0 + UPW - 2 + b
        drain_gathers(b)
        wait_store(b)
        transpose(b)
        fire_store(b, u)
    wait_store(0)
    wait_store(1)


def kernel(x, table):
    xt = jnp.transpose(x.astype(jnp.int32))
    out5 = _emb_lookup(xt, table)
    out = jnp.transpose(out5, (2, 4, 0, 1, 3))
    return out.reshape(BATCH, HIST, EMBED)
